# baseline (speedup 1.0000x reference)
import functools

import jax
import jax.numpy as jnp
import numpy as np
from jax import lax
from jax.experimental import pallas as pl
from jax.experimental.pallas import tpu as pltpu
from jax.experimental.pallas import tpu_sc as plsc

_NUM_VOCAB = 1000000
_NUM_HID = 64
_BATCH = 4096
_SEQ = 200

_NC = 2
_NS = 16
_NW = _NC * _NS

_ROWS = _BATCH * _SEQ
_ROWS_PER_W = _ROWS // _NW
_CHUNK = 512
_GATHER = 128
_NG = _CHUNK // _GATHER
_NCHUNK = _ROWS_PER_W // _CHUNK


def _positional_encoding() -> np.ndarray:
    depth = _NUM_HID // 2
    positions = np.arange(_SEQ, dtype=np.float32)[:, None]
    depths = np.arange(depth, dtype=np.float32)[None, :] / np.float32(depth)
    angle_rates = (1.0 / np.power(np.float32(10000.0), depths)).astype(np.float32)
    angle_rads = positions * angle_rates
    return np.concatenate(
        [np.sin(angle_rads), np.cos(angle_rads)], axis=-1).astype(np.float32)


_POS_NP = _positional_encoding()
_SCALE = float(np.sqrt(np.float32(_NUM_HID)))


def _sc_body(x_hbm, pos_hbm, table_hbm, out_hbm, idx_v, rows_v, pos_v, sem):
    wid = lax.axis_index("s") * _NC + lax.axis_index("c")
    base = wid * _ROWS_PER_W
    xrow = wid * (_ROWS_PER_W // _GATHER)

    pltpu.sync_copy(pos_hbm, pos_v)

    def chunk_body(c, _):
        pltpu.sync_copy(x_hbm.at[pl.ds(xrow + c * _NG, _NG)], idx_v)
        descs = [
            pltpu.async_copy(
                table_hbm.at[idx_v.at[g]],
                rows_v.at[pl.ds(g * _GATHER, _GATHER)],
                sem,
            )
            for g in range(_NG)
        ]
        for d in descs:
            d.wait()

        p0 = lax.rem(c * _CHUNK, _SEQ)

        def row_body(r, p):
            for j in range(_NUM_HID // 16):
                sl = pl.ds(j * 16, 16)
                rows_v[r, sl] = rows_v[r, sl] * _SCALE + pos_v[p, sl]
            p = p + 1
            return jnp.where(p >= _SEQ, 0, p)

        lax.fori_loop(0, _CHUNK, row_body, p0, unroll=2)

        pltpu.sync_copy(rows_v, out_hbm.at[pl.ds(base + c * _CHUNK, _CHUNK)])
        return ()

    lax.fori_loop(0, _NCHUNK, chunk_body, ())


@functools.partial(jax.jit, static_argnames=())
def _run(x_flat2d, pos, emb_table):
    mesh = plsc.VectorSubcoreMesh(core_axis_name="c", subcore_axis_name="s")
    f = functools.partial(
        pl.kernel,
        mesh=mesh,
        out_type=jax.ShapeDtypeStruct((_ROWS, _NUM_HID), jnp.float32),
        scratch_types=[
            pltpu.VMEM((_NG, _GATHER), jnp.int32),
            pltpu.VMEM((_CHUNK, _NUM_HID), jnp.float32),
            pltpu.VMEM((_SEQ, _NUM_HID), jnp.float32),
            pltpu.SemaphoreType.DMA,
        ],
        compiler_params=pltpu.CompilerParams(use_tc_tiling_on_sc=False),
    )(_sc_body)
    return f(x_flat2d, pos, emb_table)


def kernel(x, emb_table):
    x_flat2d = x.reshape(_ROWS // _GATHER, _GATHER)
    pos = jnp.asarray(_POS_NP)
    out = _run(x_flat2d, pos, emb_table)
    return out.reshape(_BATCH, _SEQ, _NUM_HID)
